# baseline (speedup 1.0000x reference)
import functools

import jax
import jax.numpy as jnp
from jax import lax
from jax.experimental import pallas as pl
from jax.experimental.pallas import tpu as pltpu
from jax.experimental.pallas import tpu_sc as plsc

B, D = 4096, 12288
TBL_PAD = 1024

_NC, _NS = 2, 16
_NW = _NC * _NS
_CHUNK = B // _NW
_LANES = 16


def _make_sc_gather():
    mesh = plsc.VectorSubcoreMesh(core_axis_name="c", subcore_axis_name="s")

    @functools.partial(
        pl.kernel,
        mesh=mesh,
        out_type=(
            jax.ShapeDtypeStruct((B,), jnp.float32),
            jax.ShapeDtypeStruct((B,), jnp.float32),
        ),
        scratch_types=[
            pltpu.VMEM((_CHUNK,), jnp.int32),
            pltpu.VMEM((_CHUNK,), jnp.float32),
            pltpu.VMEM((_CHUNK,), jnp.float32),
            pltpu.SemaphoreType.DMA,
            pltpu.SemaphoreType.DMA,
        ],
    )
    def sc_gather(t_hbm, sa_hbm, sb_hbm, oa_hbm, ob_hbm,
                  idx_v, oa_v, ob_v, sem_a, sem_b):
        wid = lax.axis_index("s") * _NC + lax.axis_index("c")
        base = wid * _CHUNK
        pltpu.sync_copy(t_hbm.at[pl.ds(base, _CHUNK)], idx_v)
        cp_a = pltpu.async_copy(sa_hbm.at[idx_v], oa_v, sem_a)
        cp_b = pltpu.async_copy(sb_hbm.at[idx_v], ob_v, sem_b)
        cp_a.wait()
        cp_b.wait()
        pltpu.sync_copy(oa_v, oa_hbm.at[pl.ds(base, _CHUNK)])
        pltpu.sync_copy(ob_v, ob_hbm.at[pl.ds(base, _CHUNK)])

    return sc_gather


_BR = 64


def _tc_body(sa_ref, sb_ref, x_ref, n_ref, o_ref):
    o_ref[...] = jnp.clip(
        sa_ref[...] * x_ref[...] + sb_ref[...] * n_ref[...], -1.0, 1.0)


def kernel(x_0, t, noise, sqrt_alpha, sqrt_1m_alpha):
    sa_tbl = jnp.pad(sqrt_alpha, (0, TBL_PAD - sqrt_alpha.shape[0]))
    sb_tbl = jnp.pad(sqrt_1m_alpha, (0, TBL_PAD - sqrt_1m_alpha.shape[0]))
    scale_a, scale_b = _make_sc_gather()(t, sa_tbl, sb_tbl)
    out = pl.pallas_call(
        _tc_body,
        grid=(B // _BR,),
        in_specs=[
            pl.BlockSpec((_BR, 1), lambda i: (i, 0)),
            pl.BlockSpec((_BR, 1), lambda i: (i, 0)),
            pl.BlockSpec((_BR, D), lambda i: (i, 0)),
            pl.BlockSpec((_BR, D), lambda i: (i, 0)),
        ],
        out_specs=pl.BlockSpec((_BR, D), lambda i: (i, 0)),
        out_shape=jax.ShapeDtypeStruct((B, D), jnp.float32),
    )(scale_a.reshape(B, 1), scale_b.reshape(B, 1), x_0, noise)
    return out, noise


# speedup vs baseline: 1.0176x; 1.0176x over previous
import functools

import jax
import jax.numpy as jnp
from jax import lax
from jax.experimental import pallas as pl
from jax.experimental.pallas import tpu as pltpu
from jax.experimental.pallas import tpu_sc as plsc

B, D = 4096, 12288
TBL_PAD = 1024

_NC, _NS = 2, 16
_NW = _NC * _NS
_CHUNK = B // _NW
_LANES = 16


def _make_sc_gather():
    mesh = plsc.VectorSubcoreMesh(core_axis_name="c", subcore_axis_name="s")

    @functools.partial(
        pl.kernel,
        mesh=mesh,
        out_type=(
            jax.ShapeDtypeStruct((B,), jnp.float32),
            jax.ShapeDtypeStruct((B,), jnp.float32),
        ),
        scratch_types=[
            pltpu.VMEM((_CHUNK,), jnp.int32),
            pltpu.VMEM((_CHUNK,), jnp.float32),
            pltpu.VMEM((_CHUNK,), jnp.float32),
            pltpu.SemaphoreType.DMA,
            pltpu.SemaphoreType.DMA,
        ],
    )
    def sc_gather(t_hbm, sa_hbm, sb_hbm, oa_hbm, ob_hbm,
                  idx_v, oa_v, ob_v, sem_a, sem_b):
        wid = lax.axis_index("s") * _NC + lax.axis_index("c")
        base = wid * _CHUNK
        pltpu.sync_copy(t_hbm.at[pl.ds(base, _CHUNK)], idx_v)
        cp_a = pltpu.async_copy(sa_hbm.at[idx_v], oa_v, sem_a)
        cp_b = pltpu.async_copy(sb_hbm.at[idx_v], ob_v, sem_b)
        cp_a.wait()
        cp_b.wait()
        pltpu.sync_copy(oa_v, oa_hbm.at[pl.ds(base, _CHUNK)])
        pltpu.sync_copy(ob_v, ob_hbm.at[pl.ds(base, _CHUNK)])

    return sc_gather


_BR = 256
_BC = 6144


def _tc_body(sa_ref, sb_ref, x_ref, n_ref, o_ref):
    o_ref[...] = jnp.clip(
        sa_ref[...] * x_ref[...] + sb_ref[...] * n_ref[...], -1.0, 1.0)


def kernel(x_0, t, noise, sqrt_alpha, sqrt_1m_alpha):
    sa_tbl = jnp.pad(sqrt_alpha, (0, TBL_PAD - sqrt_alpha.shape[0]))
    sb_tbl = jnp.pad(sqrt_1m_alpha, (0, TBL_PAD - sqrt_1m_alpha.shape[0]))
    scale_a, scale_b = _make_sc_gather()(t, sa_tbl, sb_tbl)
    out = pl.pallas_call(
        _tc_body,
        grid=(B // _BR, D // _BC),
        in_specs=[
            pl.BlockSpec((_BR, 1), lambda i, j: (i, 0)),
            pl.BlockSpec((_BR, 1), lambda i, j: (i, 0)),
            pl.BlockSpec((_BR, _BC), lambda i, j: (i, j)),
            pl.BlockSpec((_BR, _BC), lambda i, j: (i, j)),
        ],
        out_specs=pl.BlockSpec((_BR, _BC), lambda i, j: (i, j)),
        out_shape=jax.ShapeDtypeStruct((B, D), jnp.float32),
    )(scale_a.reshape(B, 1), scale_b.reshape(B, 1), x_0, noise)
    return out, noise
